# baseline (speedup 1.0000x reference)
import functools

import jax
import jax.numpy as jnp
from jax import lax
from jax.experimental import pallas as pl
from jax.experimental.pallas import tpu as pltpu
from jax.experimental.pallas import tpu_sc as plsc

_B, _S, _D = 4, 4096, 1024
_NC, _NS = 2, 16
_NW = _NC * _NS
_ROWS_PER_W = _S // _NW
_SUB_ROWS = 32
_N_SUB = _ROWS_PER_W // _SUB_ROWS
_CHUNK = _SUB_ROWS * _D


def _sc_body(x_hbm, p_hbm, o_hbm, x_v, p_v):
    wid = lax.axis_index("s") * _NC + lax.axis_index("c")
    row0 = wid * _ROWS_PER_W
    for t in range(_N_SUB):
        off = (row0 + t * _SUB_ROWS) * _D
        pltpu.sync_copy(p_hbm.at[pl.ds(off, _CHUNK)], p_v)
        for b in range(_B):
            pltpu.sync_copy(x_hbm.at[b, pl.ds(off, _CHUNK)], x_v)

            def add_body(i, _):
                sl = pl.ds(i * 16, 16)
                x_v[sl] = x_v[sl] + p_v[sl]
                return 0

            lax.fori_loop(0, _CHUNK // 16, add_body, 0)
            pltpu.sync_copy(x_v, o_hbm.at[b, pl.ds(off, _CHUNK)])


def kernel(inputs, position_embeddings):
    B, S, D = inputs.shape
    x = inputs.reshape(B, S * D)
    p = position_embeddings.reshape(-1)
    mesh = plsc.VectorSubcoreMesh(core_axis_name="c", subcore_axis_name="s")
    out = pl.kernel(
        _sc_body,
        out_type=jax.ShapeDtypeStruct((B, S * D), jnp.float32),
        mesh=mesh,
        scratch_types=[
            pltpu.VMEM((_CHUNK,), jnp.float32),
            pltpu.VMEM((_CHUNK,), jnp.float32),
        ],
    )(x, p)
    return out.reshape(B, S, D)


# speedup vs baseline: 1.6838x; 1.6838x over previous
import functools

import jax
import jax.numpy as jnp
from jax import lax
from jax.experimental import pallas as pl
from jax.experimental.pallas import tpu as pltpu
from jax.experimental.pallas import tpu_sc as plsc

_B, _S, _D = 4, 4096, 1024
_NC, _NS = 2, 16
_NW = _NC * _NS
_ROWS_PER_W = _S // _NW
_SUB_ROWS = 32
_N_SUB = _ROWS_PER_W // _SUB_ROWS
_CHUNK = _SUB_ROWS * _D
_NTASK = _N_SUB * _B


def _sc_body(x_hbm, p_hbm, o_hbm, xv0, xv1, pv, sin0, sin1, sout0, sout1):
    wid = lax.axis_index("s") * _NC + lax.axis_index("c")
    row0 = wid * _ROWS_PER_W
    xbufs = (xv0, xv1)
    sins = (sin0, sin1)
    souts = (sout0, sout1)
    cin = [None, None]
    cout = [None, None]

    off0 = row0 * _D
    cin[0] = pltpu.async_copy(x_hbm.at[0, pl.ds(off0, _CHUNK)], xv0, sin0)
    for k in range(_NTASK):
        t, b = divmod(k, _B)
        off = (row0 + t * _SUB_ROWS) * _D
        if b == 0:
            pltpu.sync_copy(p_hbm.at[pl.ds(off, _CHUNK)], pv)
        nk = k + 1
        if nk < _NTASK:
            nt, nb = divmod(nk, _B)
            noff = (row0 + nt * _SUB_ROWS) * _D
            nbuf = nk % 2
            if cout[nbuf] is not None:
                cout[nbuf].wait()
            cin[nbuf] = pltpu.async_copy(
                x_hbm.at[nb, pl.ds(noff, _CHUNK)], xbufs[nbuf], sins[nbuf]
            )
        cb = k % 2
        cin[cb].wait()
        xbuf = xbufs[cb]

        @plsc.parallel_loop(0, _CHUNK, step=16, unroll=8)
        def add_body(i):
            plsc.addupdate(xbuf.at[pl.ds(i, 16)], pv[pl.ds(i, 16)])

        cout[cb] = pltpu.async_copy(xbuf, o_hbm.at[b, pl.ds(off, _CHUNK)], souts[cb])
    cout[0].wait()
    cout[1].wait()


def kernel(inputs, position_embeddings):
    B, S, D = inputs.shape
    x = inputs.reshape(B, S * D)
    p = position_embeddings.reshape(-1)
    mesh = plsc.VectorSubcoreMesh(core_axis_name="c", subcore_axis_name="s")
    out = pl.kernel(
        _sc_body,
        out_type=jax.ShapeDtypeStruct((B, S * D), jnp.float32),
        mesh=mesh,
        scratch_types=[
            pltpu.VMEM((_CHUNK,), jnp.float32),
            pltpu.VMEM((_CHUNK,), jnp.float32),
            pltpu.VMEM((_CHUNK,), jnp.float32),
            pltpu.SemaphoreType.DMA,
            pltpu.SemaphoreType.DMA,
            pltpu.SemaphoreType.DMA,
            pltpu.SemaphoreType.DMA,
        ],
    )(x, p)
    return out.reshape(B, S, D)


# speedup vs baseline: 1.6956x; 1.0070x over previous
import functools

import jax
import jax.numpy as jnp
from jax import lax
from jax.experimental import pallas as pl
from jax.experimental.pallas import tpu as pltpu
from jax.experimental.pallas import tpu_sc as plsc

_B, _S, _D = 4, 4096, 1024
_NC, _NS = 2, 16
_NW = _NC * _NS
_ROWS_PER_W = _S // _NW
_SUB_ROWS = 16
_N_SUB = _ROWS_PER_W // _SUB_ROWS
_CHUNK = _SUB_ROWS * _D
_NTASK = _N_SUB * _B


def _sc_body(x_hbm, p_hbm, o_hbm, xv0, xv1, ov0, ov1, pv,
             sin0, sin1, sout0, sout1):
    wid = lax.axis_index("s") * _NC + lax.axis_index("c")
    row0 = wid * _ROWS_PER_W
    xbufs = (xv0, xv1)
    obufs = (ov0, ov1)
    sins = (sin0, sin1)
    souts = (sout0, sout1)
    cin = [None, None]
    cout = [None, None]

    off0 = row0 * _D
    cin[0] = pltpu.async_copy(x_hbm.at[0, pl.ds(off0, _CHUNK)], xv0, sin0)
    for k in range(_NTASK):
        t, b = divmod(k, _B)
        off = (row0 + t * _SUB_ROWS) * _D
        if b == 0:
            pltpu.sync_copy(p_hbm.at[pl.ds(off, _CHUNK)], pv)
        nk = k + 1
        if nk < _NTASK:
            nt, nb = divmod(nk, _B)
            noff = (row0 + nt * _SUB_ROWS) * _D
            nbuf = nk % 2
            cin[nbuf] = pltpu.async_copy(
                x_hbm.at[nb, pl.ds(noff, _CHUNK)], xbufs[nbuf], sins[nbuf]
            )
        cb = k % 2
        cin[cb].wait()
        xbuf = xbufs[cb]
        obuf = obufs[cb]
        if cout[cb] is not None:
            cout[cb].wait()

        @plsc.parallel_loop(0, _CHUNK, step=16, unroll=8)
        def add_body(i):
            sl = pl.ds(i, 16)
            obuf[sl] = xbuf[sl] + pv[sl]

        cout[cb] = pltpu.async_copy(obuf, o_hbm.at[b, pl.ds(off, _CHUNK)], souts[cb])
    cout[0].wait()
    cout[1].wait()


def kernel(inputs, position_embeddings):
    B, S, D = inputs.shape
    x = inputs.reshape(B, S * D)
    p = position_embeddings.reshape(-1)
    mesh = plsc.VectorSubcoreMesh(core_axis_name="c", subcore_axis_name="s")
    out = pl.kernel(
        _sc_body,
        out_type=jax.ShapeDtypeStruct((B, S * D), jnp.float32),
        mesh=mesh,
        scratch_types=[
            pltpu.VMEM((_CHUNK,), jnp.float32),
            pltpu.VMEM((_CHUNK,), jnp.float32),
            pltpu.VMEM((_CHUNK,), jnp.float32),
            pltpu.VMEM((_CHUNK,), jnp.float32),
            pltpu.VMEM((_CHUNK,), jnp.float32),
            pltpu.SemaphoreType.DMA,
            pltpu.SemaphoreType.DMA,
            pltpu.SemaphoreType.DMA,
            pltpu.SemaphoreType.DMA,
        ],
    )(x, p)
    return out.reshape(B, S, D)
